# Initial kernel scaffold; baseline (speedup 1.0000x reference)
import jax
import jax.numpy as jnp
from jax.experimental import pallas as pl


def kernel(target_idx, context_idx, neg_samples, in_emb, out_emb):
    raise NotImplementedError("write your pallas kernel here")


# speedup vs baseline: 4.3860x; 4.3860x over previous
import functools

import jax
import jax.numpy as jnp
from jax import lax
from jax.experimental import pallas as pl
from jax.experimental.pallas import tpu as pltpu
from jax.experimental.pallas import tpu_sc as plsc

_NC = 2
_NS = 16
_LANES = 16


def _sc_scores(tgt, ctx, neg2d, in_emb, out_emb, *, B, D, NNEG, C):
    NW = _NC * _NS
    pairs_per_w = B // NW
    nchunk = pairs_per_w // C
    NSC = NNEG + 1
    SCORE_W = NSC * _LANES
    NG = (C * NNEG) // 128
    DV = D // _LANES

    mesh = plsc.VectorSubcoreMesh(
        core_axis_name="c", subcore_axis_name="s",
        num_cores=_NC, num_subcores=_NS)

    def body(tgt_h, ctx_h, neg_h, ine_h, oute_h, out_h,
             tidx, cidx, nidx, trows, crows, nrows, outv, sem):
        wid = lax.axis_index("s") * _NC + lax.axis_index("c")

        def chunk(g, carry):
            base = (wid * nchunk + g) * C
            pltpu.sync_copy(tgt_h.at[pl.ds(base, C)], tidx)
            pltpu.sync_copy(ctx_h.at[pl.ds(base, C)], cidx)
            pltpu.sync_copy(neg_h.at[pl.ds(base * NNEG, C * NNEG)], nidx)
            cps = [pltpu.async_copy(ine_h.at[tidx], trows, sem),
                   pltpu.async_copy(oute_h.at[cidx], crows, sem)]
            for j in range(NG):
                cps.append(pltpu.async_copy(
                    oute_h.at[nidx.at[pl.ds(j * 128, 128)]],
                    nrows.at[pl.ds(j * 128, 128)], sem))
            for cp in cps:
                cp.wait()

            def pair(p, c2):
                t = [trows[p, pl.ds(d * _LANES, _LANES)] for d in range(DV)]
                acc = t[0] * crows[p, pl.ds(0, _LANES)]
                for d in range(1, DV):
                    acc = acc + t[d] * crows[p, pl.ds(d * _LANES, _LANES)]
                outv[pl.ds(p * SCORE_W, _LANES)] = acc
                for n in range(NNEG):
                    r = p * NNEG + n
                    nacc = t[0] * nrows[r, pl.ds(0, _LANES)]
                    for d in range(1, DV):
                        nacc = nacc + t[d] * nrows[r, pl.ds(d * _LANES, _LANES)]
                    outv[pl.ds(p * SCORE_W + (n + 1) * _LANES, _LANES)] = nacc
                return c2

            lax.fori_loop(0, C, pair, 0)
            pltpu.sync_copy(outv, out_h.at[pl.ds(base * SCORE_W, C * SCORE_W)])
            return carry

        lax.fori_loop(0, nchunk, chunk, 0)

    return pl.kernel(
        body,
        out_type=jax.ShapeDtypeStruct((B * SCORE_W,), jnp.float32),
        mesh=mesh,
        scratch_types=[
            pltpu.VMEM((C,), jnp.int32),
            pltpu.VMEM((C,), jnp.int32),
            pltpu.VMEM((C * NNEG,), jnp.int32),
            pltpu.VMEM((C, D), jnp.float32),
            pltpu.VMEM((C, D), jnp.float32),
            pltpu.VMEM((C * NNEG, D), jnp.float32),
            pltpu.VMEM((C * SCORE_W,), jnp.float32),
            pltpu.SemaphoreType.DMA,
        ],
        compiler_params=pltpu.CompilerParams(use_tc_tiling_on_sc=False),
    )(tgt, ctx, neg2d, in_emb, out_emb)


def _tc_loss(partials2d, *, B, NNEG, nblocks):
    NSC = NNEG + 1
    W = NSC * _LANES
    rows = B // nblocks

    def body(pref, oref):
        i = pl.program_id(0)
        x = pref[...]
        pos = jnp.sum(x[:, 0:_LANES], axis=1)
        tot = jax.nn.log_sigmoid(pos)
        for n in range(1, NSC):
            s = jnp.sum(x[:, n * _LANES:(n + 1) * _LANES], axis=1)
            tot = tot + jax.nn.log_sigmoid(-s)

        @pl.when(i == 0)
        def _():
            oref[0, 0] = 0.0

        oref[0, 0] = oref[0, 0] + jnp.sum(tot) * (-1.0 / B)

    return pl.pallas_call(
        body,
        grid=(nblocks,),
        in_specs=[pl.BlockSpec((rows, W), lambda i: (i, 0))],
        out_specs=pl.BlockSpec(memory_space=pltpu.SMEM),
        out_shape=jax.ShapeDtypeStruct((1, 1), jnp.float32),
    )(partials2d)


def kernel(target_idx, context_idx, neg_samples, in_emb, out_emb):
    B = target_idx.shape[0]
    _, D = in_emb.shape
    NNEG = neg_samples.shape[1]
    tgt = target_idx.astype(jnp.int32)
    ctx = context_idx.astype(jnp.int32)
    neg2d = neg_samples.astype(jnp.int32).reshape(B * NNEG)
    C = 64
    partials = _sc_scores(tgt, ctx, neg2d, in_emb, out_emb,
                          B=B, D=D, NNEG=NNEG, C=C)
    partials2d = partials.reshape(B, (NNEG + 1) * _LANES)
    loss = _tc_loss(partials2d, B=B, NNEG=NNEG, nblocks=32)
    return loss[0, 0]


# speedup vs baseline: 10.3485x; 2.3594x over previous
import functools

import jax
import jax.numpy as jnp
from jax import lax
from jax.experimental import pallas as pl
from jax.experimental.pallas import tpu as pltpu
from jax.experimental.pallas import tpu_sc as plsc

_NC = 2
_NS = 16
_LANES = 16


def _sc_scores(tgt, ctx, neg2d, in_emb, out_emb, *, B, D, NNEG, C, SPLIT):
    NW = _NC * _NS
    pairs_per_w = B // NW
    nchunk = pairs_per_w // C
    NSC = NNEG + 1
    SCORE_W = NSC * _LANES
    NG = (C * NNEG) // 128
    DV = D // _LANES

    mesh = plsc.VectorSubcoreMesh(
        core_axis_name="c", subcore_axis_name="s",
        num_cores=_NC, num_subcores=_NS)

    OUTW = C * SCORE_W
    NSTEP = nchunk // 2

    def body(tgt_h, ctx_h, neg_h, ine_h, oute_h, out_h,
             ti0, ci0, ni0, tr0, cr0, nr0, ov0,
             ti1, ci1, ni1, tr1, cr1, nr1, ov1,
             semI0, semG0, semO0, semI1, semG1, semO1):
        wid = lax.axis_index("s") * _NC + lax.axis_index("c")
        bufs = ((ti0, ci0, ni0, tr0, cr0, nr0, ov0, semI0, semG0, semO0),
                (ti1, ci1, ni1, tr1, cr1, nr1, ov1, semI1, semG1, semO1))

        def remap(ref, n):
            def f(k, c2):
                v = ref[pl.ds(k * _LANES, _LANES)]
                ref[pl.ds(k * _LANES, _LANES)] = jnp.where(
                    v < SPLIT, 2 * v, 2 * (v - SPLIT) + 1)
                return c2

            lax.fori_loop(0, n // _LANES, f, 0)

        def fire_idx(g, b):
            ti, ci, ni = bufs[b][0], bufs[b][1], bufs[b][2]
            base = (wid * nchunk + g) * C
            pltpu.async_copy(tgt_h.at[pl.ds(base, C)], ti, bufs[b][7])
            pltpu.async_copy(ctx_h.at[pl.ds(base, C)], ci, bufs[b][7])
            pltpu.async_copy(neg_h.at[pl.ds(base * NNEG, C * NNEG)], ni,
                             bufs[b][7])

        def drain_idx(b):
            pltpu.make_async_copy(tgt_h.at[pl.ds(0, C)], bufs[b][0],
                                  bufs[b][7]).wait()
            pltpu.make_async_copy(ctx_h.at[pl.ds(0, C)], bufs[b][1],
                                  bufs[b][7]).wait()
            pltpu.make_async_copy(neg_h.at[pl.ds(0, C * NNEG)], bufs[b][2],
                                  bufs[b][7]).wait()

        def fire_gathers(b):
            ti, ci, ni, tr, cr, nr = bufs[b][:6]
            pltpu.async_copy(ine_h.at[ti], tr, bufs[b][8])
            pltpu.async_copy(oute_h.at[ci], cr, bufs[b][8])
            for j in range(NG):
                pltpu.async_copy(oute_h.at[ni.at[pl.ds(j * 128, 128)]],
                                 nr.at[pl.ds(j * 128, 128)], bufs[b][8])

        def drain_gathers(b):
            pltpu.make_async_copy(ine_h.at[pl.ds(0, C)], bufs[b][3],
                                  bufs[b][8]).wait()
            pltpu.make_async_copy(oute_h.at[pl.ds(0, C)], bufs[b][4],
                                  bufs[b][8]).wait()
            pltpu.make_async_copy(oute_h.at[pl.ds(0, C * NNEG)], bufs[b][5],
                                  bufs[b][8]).wait()

        def stage(b):
            remap(bufs[b][0], C)
            remap(bufs[b][1], C)
            remap(bufs[b][2], C * NNEG)

        def compute(g, b):
            tr, cr, nr, ov = bufs[b][3], bufs[b][4], bufs[b][5], bufs[b][6]

            def pair(p, c2):
                t = [tr[p, pl.ds(d * _LANES, _LANES)] for d in range(DV)]
                acc = t[0] * cr[p, pl.ds(0, _LANES)]
                for d in range(1, DV):
                    acc = acc + t[d] * cr[p, pl.ds(d * _LANES, _LANES)]
                ov[pl.ds(p * SCORE_W, _LANES)] = acc
                for n in range(NNEG):
                    r = p * NNEG + n
                    nacc = t[0] * nr[r, pl.ds(0, _LANES)]
                    for d in range(1, DV):
                        nacc = nacc + t[d] * nr[r, pl.ds(d * _LANES, _LANES)]
                    ov[pl.ds(p * SCORE_W + (n + 1) * _LANES, _LANES)] = nacc
                return c2

            lax.fori_loop(0, C, pair, 0)
            base = (wid * nchunk + g) * C
            pltpu.async_copy(ov, out_h.at[pl.ds(base * SCORE_W, OUTW)],
                             bufs[b][9])

        def drain_out(b):
            pltpu.make_async_copy(bufs[b][6], out_h.at[pl.ds(0, OUTW)],
                                  bufs[b][9]).wait()

        fire_idx(0, 0)
        fire_idx(1, 1)
        drain_idx(0)
        stage(0)
        fire_gathers(0)

        def step(s, carry):
            a = 2 * s
            drain_idx(1)
            stage(1)
            fire_gathers(1)
            drain_gathers(0)

            @pl.when(s + 1 < NSTEP)
            def _():
                fire_idx(a + 2, 0)

            @pl.when(s >= 1)
            def _():
                drain_out(0)

            compute(a, 0)

            @pl.when(s + 1 < NSTEP)
            def _():
                drain_idx(0)
                stage(0)
                fire_gathers(0)

            drain_gathers(1)

            @pl.when(s + 1 < NSTEP)
            def _():
                fire_idx(a + 3, 1)

            @pl.when(s >= 1)
            def _():
                drain_out(1)

            compute(a + 1, 1)
            return carry

        lax.fori_loop(0, NSTEP, step, 0)
        drain_out(0)
        drain_out(1)

    return pl.kernel(
        body,
        out_type=jax.ShapeDtypeStruct((B * SCORE_W,), jnp.float32),
        mesh=mesh,
        scratch_types=(
            [pltpu.VMEM((C,), jnp.int32),
             pltpu.VMEM((C,), jnp.int32),
             pltpu.VMEM((C * NNEG,), jnp.int32),
             pltpu.VMEM((C, D), jnp.float32),
             pltpu.VMEM((C, D), jnp.float32),
             pltpu.VMEM((C * NNEG, D), jnp.float32),
             pltpu.VMEM((C * SCORE_W,), jnp.float32)] * 2
            + [pltpu.SemaphoreType.DMA] * 6),
        compiler_params=pltpu.CompilerParams(use_tc_tiling_on_sc=False),
    )(tgt, ctx, neg2d, in_emb, out_emb)


def _tc_transpose(tab_t, *, V, D, BR):
    S = (V // 2) // BR * BR
    nb2 = -(-(V - S) // BR)
    H = nb2 * BR
    off = S // BR

    def body(x1_ref, x2_ref, o_ref):
        o_ref[...] = jnp.concatenate(
            [x1_ref[...].T, x2_ref[...].T], axis=1)

    out = pl.pallas_call(
        body,
        grid=(nb2,),
        in_specs=[pl.BlockSpec((D, BR), lambda i: (0, i)),
                  pl.BlockSpec((D, BR), lambda i: (0, i + off))],
        out_specs=pl.BlockSpec((BR, 2 * D), lambda i: (i, 0)),
        out_shape=jax.ShapeDtypeStruct((H, 2 * D), jnp.float32),
        compiler_params=pltpu.CompilerParams(vmem_limit_bytes=100_000_000),
    )(tab_t, tab_t)
    return out, S, H


def _tc_loss(partials2d, *, B, NNEG, nblocks):
    NSC = NNEG + 1
    R = partials2d.shape[0]
    rows = R // nblocks

    def body(pref, oref):
        i = pl.program_id(0)
        x = pref[...]
        rr = lax.broadcasted_iota(jnp.int32, (128, 8), 0)
        cc = lax.broadcasted_iota(jnp.int32, (128, 8), 1)
        m = (rr // _LANES == cc).astype(jnp.float32)
        s = jnp.dot(x, m, preferred_element_type=jnp.float32)
        gr = lax.broadcasted_iota(jnp.int32, (rows, 8), 0)
        gc = lax.broadcasted_iota(jnp.int32, (rows, 8), 1)
        g = (i * rows + gr) * 8 + gc
        j = g % NSC
        val = jnp.where(j == 0, s, -s)
        tot = jnp.sum(jax.nn.log_sigmoid(val))

        @pl.when(i == 0)
        def _():
            oref[0, 0] = 0.0

        oref[0, 0] = oref[0, 0] + tot * (-1.0 / B)

    return pl.pallas_call(
        body,
        grid=(nblocks,),
        in_specs=[pl.BlockSpec((rows, 128), lambda i: (i, 0))],
        out_specs=pl.BlockSpec(memory_space=pltpu.SMEM),
        out_shape=jax.ShapeDtypeStruct((1, 1), jnp.float32),
    )(partials2d)


def kernel(target_idx, context_idx, neg_samples, in_emb, out_emb):
    B = target_idx.shape[0]
    _, D = in_emb.shape
    NNEG = neg_samples.shape[1]
    tgt = target_idx.astype(jnp.int32)
    ctx = context_idx.astype(jnp.int32)
    neg2d = neg_samples.astype(jnp.int32).reshape(B * NNEG)
    C = 32
    V = in_emb.shape[0]
    in_p, S, H = _tc_transpose(in_emb.T, V=V, D=D, BR=16384)
    out_p, _, _ = _tc_transpose(out_emb.T, V=V, D=D, BR=16384)
    in_rm = in_p.reshape(-1).reshape(2 * H, D)
    out_rm = out_p.reshape(-1).reshape(2 * H, D)
    partials = _sc_scores(tgt, ctx, neg2d, in_rm, out_rm,
                          B=B, D=D, NNEG=NNEG, C=C, SPLIT=S)
    partials2d = partials.reshape(B * (NNEG + 1) * _LANES // 128, 128)
    loss = _tc_loss(partials2d, B=B, NNEG=NNEG, nblocks=32)
    return loss[0, 0]


# speedup vs baseline: 10.4594x; 1.0107x over previous
import functools

import jax
import jax.numpy as jnp
from jax import lax
from jax.experimental import pallas as pl
from jax.experimental.pallas import tpu as pltpu
from jax.experimental.pallas import tpu_sc as plsc

_NC = 2
_NS = 16
_LANES = 16


def _sc_scores(tgt, ctx, neg2d, in_emb, out_emb, *, B, D, NNEG, C, SPLIT):
    NW = _NC * _NS
    pairs_per_w = B // NW
    nchunk = pairs_per_w // C
    NSC = NNEG + 1
    SCORE_W = NSC * _LANES
    NG = (C * NNEG) // 128
    DV = D // _LANES

    mesh = plsc.VectorSubcoreMesh(
        core_axis_name="c", subcore_axis_name="s",
        num_cores=_NC, num_subcores=_NS)

    OUTW = C * SCORE_W
    NSTEP = nchunk // 2

    def body(tgt_h, ctx_h, neg_h, ine_h, oute_h, out_h,
             ti0, ci0, ni0, tr0, cr0, nr0, ov0,
             ti1, ci1, ni1, tr1, cr1, nr1, ov1,
             semI0, semG0, semO0, semI1, semG1, semO1):
        wid = lax.axis_index("s") * _NC + lax.axis_index("c")
        bufs = ((ti0, ci0, ni0, tr0, cr0, nr0, ov0, semI0, semG0, semO0),
                (ti1, ci1, ni1, tr1, cr1, nr1, ov1, semI1, semG1, semO1))

        def remap(ref, n):
            def f(k, c2):
                v = ref[pl.ds(k * _LANES, _LANES)]
                ref[pl.ds(k * _LANES, _LANES)] = jnp.where(
                    v < SPLIT, 2 * v, 2 * (v - SPLIT) + 1)
                return c2

            lax.fori_loop(0, n // _LANES, f, 0)

        def fire_idx(g, b):
            ti, ci, ni = bufs[b][0], bufs[b][1], bufs[b][2]
            base = (wid * nchunk + g) * C
            pltpu.async_copy(tgt_h.at[pl.ds(base, C)], ti, bufs[b][7])
            pltpu.async_copy(ctx_h.at[pl.ds(base, C)], ci, bufs[b][7])
            pltpu.async_copy(neg_h.at[pl.ds(base * NNEG, C * NNEG)], ni,
                             bufs[b][7])

        def drain_idx(b):
            pltpu.make_async_copy(tgt_h.at[pl.ds(0, C)], bufs[b][0],
                                  bufs[b][7]).wait()
            pltpu.make_async_copy(ctx_h.at[pl.ds(0, C)], bufs[b][1],
                                  bufs[b][7]).wait()
            pltpu.make_async_copy(neg_h.at[pl.ds(0, C * NNEG)], bufs[b][2],
                                  bufs[b][7]).wait()

        def fire_gathers(b):
            ti, ci, ni, tr, cr, nr = bufs[b][:6]
            pltpu.async_copy(ine_h.at[ti], tr, bufs[b][8])
            pltpu.async_copy(oute_h.at[ci], cr, bufs[b][8])
            for j in range(NG):
                pltpu.async_copy(oute_h.at[ni.at[pl.ds(j * 128, 128)]],
                                 nr.at[pl.ds(j * 128, 128)], bufs[b][8])

        def drain_gathers(b):
            pltpu.make_async_copy(ine_h.at[pl.ds(0, C)], bufs[b][3],
                                  bufs[b][8]).wait()
            pltpu.make_async_copy(oute_h.at[pl.ds(0, C)], bufs[b][4],
                                  bufs[b][8]).wait()
            pltpu.make_async_copy(oute_h.at[pl.ds(0, C * NNEG)], bufs[b][5],
                                  bufs[b][8]).wait()

        def stage(b):
            remap(bufs[b][0], C)
            remap(bufs[b][1], C)
            remap(bufs[b][2], C * NNEG)

        def compute(g, b):
            tr, cr, nr, ov = bufs[b][3], bufs[b][4], bufs[b][5], bufs[b][6]

            def pair(p, c2):
                t = [tr[p, pl.ds(d * _LANES, _LANES)] for d in range(DV)]
                acc = t[0] * cr[p, pl.ds(0, _LANES)]
                for d in range(1, DV):
                    acc = acc + t[d] * cr[p, pl.ds(d * _LANES, _LANES)]
                ov[pl.ds(p * SCORE_W, _LANES)] = acc
                for n in range(NNEG):
                    r = p * NNEG + n
                    nacc = t[0] * nr[r, pl.ds(0, _LANES)]
                    for d in range(1, DV):
                        nacc = nacc + t[d] * nr[r, pl.ds(d * _LANES, _LANES)]
                    ov[pl.ds(p * SCORE_W + (n + 1) * _LANES, _LANES)] = nacc
                return c2

            lax.fori_loop(0, C, pair, 0)
            base = (wid * nchunk + g) * C
            pltpu.async_copy(ov, out_h.at[pl.ds(base * SCORE_W, OUTW)],
                             bufs[b][9])

        def drain_out(b):
            pltpu.make_async_copy(bufs[b][6], out_h.at[pl.ds(0, OUTW)],
                                  bufs[b][9]).wait()

        fire_idx(0, 0)
        fire_idx(1, 1)
        drain_idx(0)
        stage(0)
        fire_gathers(0)

        def step(s, carry):
            a = 2 * s
            drain_idx(1)
            stage(1)
            fire_gathers(1)
            drain_gathers(0)

            @pl.when(s + 1 < NSTEP)
            def _():
                fire_idx(a + 2, 0)

            @pl.when(s >= 1)
            def _():
                drain_out(0)

            compute(a, 0)

            @pl.when(s + 1 < NSTEP)
            def _():
                drain_idx(0)
                stage(0)
                fire_gathers(0)

            drain_gathers(1)

            @pl.when(s + 1 < NSTEP)
            def _():
                fire_idx(a + 3, 1)

            @pl.when(s >= 1)
            def _():
                drain_out(1)

            compute(a + 1, 1)
            return carry

        lax.fori_loop(0, NSTEP, step, 0)
        drain_out(0)
        drain_out(1)

    return pl.kernel(
        body,
        out_type=jax.ShapeDtypeStruct((B * SCORE_W,), jnp.float32),
        mesh=mesh,
        scratch_types=(
            [pltpu.VMEM((C,), jnp.int32),
             pltpu.VMEM((C,), jnp.int32),
             pltpu.VMEM((C * NNEG,), jnp.int32),
             pltpu.VMEM((C, D), jnp.float32),
             pltpu.VMEM((C, D), jnp.float32),
             pltpu.VMEM((C * NNEG, D), jnp.float32),
             pltpu.VMEM((C * SCORE_W,), jnp.float32)] * 2
            + [pltpu.SemaphoreType.DMA] * 6),
        compiler_params=pltpu.CompilerParams(use_tc_tiling_on_sc=False),
    )(tgt, ctx, neg2d, in_emb, out_emb)


def _tc_transpose(tab_t, *, V, D, BR):
    S = (V // 2) // BR * BR
    nb2 = -(-(V - S) // BR)
    H = nb2 * BR
    off = S // BR

    def body(x1_ref, x2_ref, o_ref):
        o_ref[...] = jnp.concatenate(
            [x1_ref[...].T, x2_ref[...].T], axis=1)

    out = pl.pallas_call(
        body,
        grid=(nb2,),
        in_specs=[pl.BlockSpec((D, BR), lambda i: (0, i)),
                  pl.BlockSpec((D, BR), lambda i: (0, i + off))],
        out_specs=pl.BlockSpec((BR, 2 * D), lambda i: (i, 0)),
        out_shape=jax.ShapeDtypeStruct((H, 2 * D), jnp.float32),
        compiler_params=pltpu.CompilerParams(vmem_limit_bytes=100_000_000),
    )(tab_t, tab_t)
    return out, S, H


def _tc_loss(partials2d, *, B, NNEG, nblocks):
    NSC = NNEG + 1
    R = partials2d.shape[0]
    rows = R // nblocks

    def body(pref, oref):
        i = pl.program_id(0)
        x = pref[...]
        rr = lax.broadcasted_iota(jnp.int32, (128, 8), 0)
        cc = lax.broadcasted_iota(jnp.int32, (128, 8), 1)
        m = (rr // _LANES == cc).astype(jnp.float32)
        s = jnp.dot(x, m, preferred_element_type=jnp.float32)
        gr = lax.broadcasted_iota(jnp.int32, (rows, 8), 0)
        gc = lax.broadcasted_iota(jnp.int32, (rows, 8), 1)
        g = (i * rows + gr) * 8 + gc
        j = g % NSC
        val = jnp.where(j == 0, s, -s)
        tot = jnp.sum(jax.nn.log_sigmoid(val))

        @pl.when(i == 0)
        def _():
            oref[0, 0] = 0.0

        oref[0, 0] = oref[0, 0] + tot * (-1.0 / B)

    return pl.pallas_call(
        body,
        grid=(nblocks,),
        in_specs=[pl.BlockSpec((rows, 128), lambda i: (i, 0))],
        out_specs=pl.BlockSpec(memory_space=pltpu.SMEM),
        out_shape=jax.ShapeDtypeStruct((1, 1), jnp.float32),
    )(partials2d)


def kernel(target_idx, context_idx, neg_samples, in_emb, out_emb):
    B = target_idx.shape[0]
    _, D = in_emb.shape
    NNEG = neg_samples.shape[1]
    tgt = target_idx.astype(jnp.int32)
    ctx = context_idx.astype(jnp.int32)
    neg2d = neg_samples.astype(jnp.int32).reshape(B * NNEG)
    C = 32
    V = in_emb.shape[0]
    in_p, S, H = _tc_transpose(in_emb.T, V=V, D=D, BR=16384)
    out_p, _, _ = _tc_transpose(out_emb.T, V=V, D=D, BR=16384)
    in_rm = in_p.reshape(-1).reshape(2 * H, D)
    out_rm = out_p.reshape(-1).reshape(2 * H, D)
    partials = _sc_scores(tgt, ctx, neg2d, in_rm, out_rm,
                          B=B, D=D, NNEG=NNEG, C=C, SPLIT=S)
    partials2d = partials.reshape(B * (NNEG + 1) * _LANES // 128, 128)
    loss = _tc_loss(partials2d, B=B, NNEG=NNEG, nblocks=8)
    return loss[0, 0]


# speedup vs baseline: 10.4598x; 1.0000x over previous
import jax
import jax.numpy as jnp
from jax import lax
from jax.experimental import pallas as pl
from jax.experimental.pallas import tpu as pltpu
from jax.experimental.pallas import tpu_sc as plsc

_NC = 2
_NS = 16
_LANES = 16


def _sc_scores(tgt, ctx, neg2d, in_emb, out_emb, *, B, D, NNEG, C, SPLIT):
    NW = _NC * _NS
    pairs_per_w = B // NW
    nchunk = pairs_per_w // C
    NSC = NNEG + 1
    SCORE_W = NSC * _LANES
    NG = (C * NNEG) // 128
    DV = D // _LANES

    mesh = plsc.VectorSubcoreMesh(
        core_axis_name="c", subcore_axis_name="s",
        num_cores=_NC, num_subcores=_NS)

    OUTW = C * SCORE_W
    NSTEP = nchunk // 2

    def body(tgt_h, ctx_h, neg_h, ine_h, oute_h, out_h,
             ti0, ci0, ni0, tr0, cr0, nr0, ov0,
             ti1, ci1, ni1, tr1, cr1, nr1, ov1,
             semI0, semG0, semO0, semI1, semG1, semO1):
        wid = lax.axis_index("s") * _NC + lax.axis_index("c")
        bufs = ((ti0, ci0, ni0, tr0, cr0, nr0, ov0, semI0, semG0, semO0),
                (ti1, ci1, ni1, tr1, cr1, nr1, ov1, semI1, semG1, semO1))

        def remap(ref, n):
            def f(k, c2):
                v = ref[pl.ds(k * _LANES, _LANES)]
                ref[pl.ds(k * _LANES, _LANES)] = jnp.where(
                    v < SPLIT, 2 * v, 2 * (v - SPLIT) + 1)
                return c2

            lax.fori_loop(0, n // _LANES, f, 0)

        def fire_idx(g, b):
            ti, ci, ni = bufs[b][0], bufs[b][1], bufs[b][2]
            base = (wid * nchunk + g) * C
            pltpu.async_copy(tgt_h.at[pl.ds(base, C)], ti, bufs[b][7])
            pltpu.async_copy(ctx_h.at[pl.ds(base, C)], ci, bufs[b][7])
            pltpu.async_copy(neg_h.at[pl.ds(base * NNEG, C * NNEG)], ni,
                             bufs[b][7])

        def drain_idx(b):
            pltpu.make_async_copy(tgt_h.at[pl.ds(0, C)], bufs[b][0],
                                  bufs[b][7]).wait()
            pltpu.make_async_copy(ctx_h.at[pl.ds(0, C)], bufs[b][1],
                                  bufs[b][7]).wait()
            pltpu.make_async_copy(neg_h.at[pl.ds(0, C * NNEG)], bufs[b][2],
                                  bufs[b][7]).wait()

        def fire_gathers(b):
            ti, ci, ni, tr, cr, nr = bufs[b][:6]
            pltpu.async_copy(ine_h.at[ti], tr, bufs[b][8])
            pltpu.async_copy(oute_h.at[ci], cr, bufs[b][8])
            for j in range(NG):
                pltpu.async_copy(oute_h.at[ni.at[pl.ds(j * 128, 128)]],
                                 nr.at[pl.ds(j * 128, 128)], bufs[b][8])

        def drain_gathers(b):
            pltpu.make_async_copy(ine_h.at[pl.ds(0, C)], bufs[b][3],
                                  bufs[b][8]).wait()
            pltpu.make_async_copy(oute_h.at[pl.ds(0, C)], bufs[b][4],
                                  bufs[b][8]).wait()
            pltpu.make_async_copy(oute_h.at[pl.ds(0, C * NNEG)], bufs[b][5],
                                  bufs[b][8]).wait()

        def stage(b):
            remap(bufs[b][0], C)
            remap(bufs[b][1], C)
            remap(bufs[b][2], C * NNEG)

        def compute(g, b):
            tr, cr, nr, ov = bufs[b][3], bufs[b][4], bufs[b][5], bufs[b][6]

            def pair(p, c2):
                t = [tr[p, pl.ds(d * _LANES, _LANES)] for d in range(DV)]
                acc = t[0] * cr[p, pl.ds(0, _LANES)]
                for d in range(1, DV):
                    acc = acc + t[d] * cr[p, pl.ds(d * _LANES, _LANES)]
                ov[pl.ds(p * SCORE_W, _LANES)] = acc
                for n in range(NNEG):
                    r = p * NNEG + n
                    nacc = t[0] * nr[r, pl.ds(0, _LANES)]
                    for d in range(1, DV):
                        nacc = nacc + t[d] * nr[r, pl.ds(d * _LANES, _LANES)]
                    ov[pl.ds(p * SCORE_W + (n + 1) * _LANES, _LANES)] = nacc
                return c2

            lax.fori_loop(0, C, pair, 0)
            base = (wid * nchunk + g) * C
            pltpu.async_copy(ov, out_h.at[pl.ds(base * SCORE_W, OUTW)],
                             bufs[b][9])

        def drain_out(b):
            pltpu.make_async_copy(bufs[b][6], out_h.at[pl.ds(0, OUTW)],
                                  bufs[b][9]).wait()

        fire_idx(0, 0)
        fire_idx(1, 1)
        drain_idx(0)
        stage(0)
        fire_gathers(0)

        def step(s, carry):
            a = 2 * s
            drain_idx(1)
            stage(1)
            fire_gathers(1)
            drain_gathers(0)

            @pl.when(s + 1 < NSTEP)
            def _():
                fire_idx(a + 2, 0)

            @pl.when(s >= 1)
            def _():
                drain_out(0)

            compute(a, 0)

            @pl.when(s + 1 < NSTEP)
            def _():
                drain_idx(0)
                stage(0)
                fire_gathers(0)

            drain_gathers(1)

            @pl.when(s + 1 < NSTEP)
            def _():
                fire_idx(a + 3, 1)

            @pl.when(s >= 1)
            def _():
                drain_out(1)

            compute(a + 1, 1)
            return carry

        lax.fori_loop(0, NSTEP, step, 0)
        drain_out(0)
        drain_out(1)

    return pl.kernel(
        body,
        out_type=jax.ShapeDtypeStruct((B * SCORE_W,), jnp.float32),
        mesh=mesh,
        scratch_types=(
            [pltpu.VMEM((C,), jnp.int32),
             pltpu.VMEM((C,), jnp.int32),
             pltpu.VMEM((C * NNEG,), jnp.int32),
             pltpu.VMEM((C, D), jnp.float32),
             pltpu.VMEM((C, D), jnp.float32),
             pltpu.VMEM((C * NNEG, D), jnp.float32),
             pltpu.VMEM((C * SCORE_W,), jnp.float32)] * 2
            + [pltpu.SemaphoreType.DMA] * 6),
        compiler_params=pltpu.CompilerParams(use_tc_tiling_on_sc=False),
    )(tgt, ctx, neg2d, in_emb, out_emb)


def _tc_transpose(tab_t, *, V, D, BR):
    S = (V // 2) // BR * BR
    nb2 = -(-(V - S) // BR)
    H = nb2 * BR
    off = S // BR

    def body(x1_ref, x2_ref, o_ref):
        o_ref[...] = jnp.concatenate(
            [x1_ref[...].T, x2_ref[...].T], axis=1)

    out = pl.pallas_call(
        body,
        grid=(nb2,),
        in_specs=[pl.BlockSpec((D, BR), lambda i: (0, i)),
                  pl.BlockSpec((D, BR), lambda i: (0, i + off))],
        out_specs=pl.BlockSpec((BR, 2 * D), lambda i: (i, 0)),
        out_shape=jax.ShapeDtypeStruct((H, 2 * D), jnp.float32),
        compiler_params=pltpu.CompilerParams(vmem_limit_bytes=100_000_000),
    )(tab_t, tab_t)
    return out, S, H


def _tc_loss(partials2d, *, B, NNEG, nblocks):
    NSC = NNEG + 1
    R = partials2d.shape[0]
    rows = R // nblocks

    def body(pref, oref):
        i = pl.program_id(0)
        x = pref[...]
        rr = lax.broadcasted_iota(jnp.int32, (128, 8), 0)
        cc = lax.broadcasted_iota(jnp.int32, (128, 8), 1)
        m = (rr // _LANES == cc).astype(jnp.float32)
        s = jnp.dot(x, m, preferred_element_type=jnp.float32)
        gr = lax.broadcasted_iota(jnp.int32, (rows, 8), 0)
        gc = lax.broadcasted_iota(jnp.int32, (rows, 8), 1)
        g = (i * rows + gr) * 8 + gc
        j = g % NSC
        val = jnp.where(j == 0, s, -s)
        tot = jnp.sum(jax.nn.log_sigmoid(val))

        @pl.when(i == 0)
        def _():
            oref[0, 0] = 0.0

        oref[0, 0] = oref[0, 0] + tot * (-1.0 / B)

    return pl.pallas_call(
        body,
        grid=(nblocks,),
        in_specs=[pl.BlockSpec((rows, 128), lambda i: (i, 0))],
        out_specs=pl.BlockSpec(memory_space=pltpu.SMEM),
        out_shape=jax.ShapeDtypeStruct((1, 1), jnp.float32),
    )(partials2d)


def kernel(target_idx, context_idx, neg_samples, in_emb, out_emb):
    B = target_idx.shape[0]
    _, D = in_emb.shape
    NNEG = neg_samples.shape[1]
    tgt = target_idx.astype(jnp.int32)
    ctx = context_idx.astype(jnp.int32)
    neg2d = neg_samples.astype(jnp.int32).reshape(B * NNEG)
    C = 32
    V = in_emb.shape[0]
    in_p, S, H = _tc_transpose(in_emb.T, V=V, D=D, BR=16384)
    out_p, _, _ = _tc_transpose(out_emb.T, V=V, D=D, BR=16384)
    in_rm = in_p.reshape(-1).reshape(2 * H, D)
    out_rm = out_p.reshape(-1).reshape(2 * H, D)
    partials = _sc_scores(tgt, ctx, neg2d, in_rm, out_rm,
                          B=B, D=D, NNEG=NNEG, C=C, SPLIT=S)
    partials2d = partials.reshape(B * (NNEG + 1) * _LANES // 128, 128)
    loss = _tc_loss(partials2d, B=B, NNEG=NNEG, nblocks=8)
    return loss[0, 0]
